# baseline (speedup 1.0000x reference)
import functools

import jax
import jax.numpy as jnp
from jax import lax
from jax.experimental import pallas as pl
from jax.experimental.pallas import tpu as pltpu
from jax.experimental.pallas import tpu_sc as plsc

N_NODES = 10000
EMB = 128
N_EDGES = 320000

NC = 2
NS = 16
NW = NC * NS
CH = 128
NCHUNK = 80
IDXSEG = 16
NSEG = NCHUNK // IDXSEG
EPW = NCHUNK * CH
E_PAD = EPW * NW
ROWS_PT = 640
ACC_ROWS = ROWS_PT * NS
DUMMY = N_NODES
DEG_W = 128

def _deg_body(dst_hbm, out_hbm, acc_sh, dst_v, ones_v, zero_v):
    c = lax.axis_index("c")
    s = lax.axis_index("s")
    wid = c * NS + s

    one16 = jnp.full((16,), 1.0, jnp.float32)
    nil16 = jnp.zeros((16,), jnp.float32)

    def _fill(i, carry):
        for j in range(DEG_W // 16):
            ones_v[i, pl.ds(j * 16, 16)] = one16
            zero_v[i, pl.ds(j * 16, 16)] = nil16
        return carry

    lax.fori_loop(0, CH, _fill, 0)

    for k in range(ROWS_PT // CH):
        pltpu.sync_copy(zero_v, acc_sh.at[pl.ds(s * ROWS_PT + k * CH, CH)])

    pltpu.sync_copy(dst_hbm.at[wid], dst_v)
    plsc.subcore_barrier()

    def _chunk(i, carry):
        pltpu.sync_copy(ones_v, acc_sh.at[dst_v.at[i]], add=True)
        return carry

    lax.fori_loop(0, NCHUNK, _chunk, 0)
    plsc.subcore_barrier()

    pltpu.sync_copy(
        acc_sh.at[pl.ds(s * ROWS_PT, ROWS_PT)],
        out_hbm.at[c, pl.ds(s * ROWS_PT, ROWS_PT)],
    )


def _agg_body(y_hbm, src_hbm, dst_hbm, out_hbm, acc_sh,
              src_v, dst_v, rows0, rows1, sem0, sem1):
    c = lax.axis_index("c")
    s = lax.axis_index("s")
    wid = c * NS + s

    nil16 = jnp.zeros((16,), jnp.float32)

    def _zero(i, carry):
        for j in range(EMB // 16):
            rows0[i, pl.ds(j * 16, 16)] = nil16
        return carry

    lax.fori_loop(0, CH, _zero, 0)

    for k in range(ROWS_PT // CH):
        pltpu.sync_copy(rows0, acc_sh.at[pl.ds(s * ROWS_PT + k * CH, CH)])

    plsc.subcore_barrier()

    def _seg(t, carry):
        pltpu.sync_copy(src_hbm.at[wid, pl.ds(t * IDXSEG, IDXSEG)], src_v)
        pltpu.sync_copy(dst_hbm.at[wid, pl.ds(t * IDXSEG, IDXSEG)], dst_v)
        pltpu.async_copy(y_hbm.at[src_v.at[0]], rows0, sem0)

        def _pair(g, carry2):
            i0 = 2 * g
            pltpu.async_copy(y_hbm.at[src_v.at[i0 + 1]], rows1, sem1)
            pltpu.make_async_copy(y_hbm.at[src_v.at[i0]], rows0, sem0).wait()
            pltpu.sync_copy(rows0, acc_sh.at[dst_v.at[i0]], add=True)
            pltpu.async_copy(y_hbm.at[src_v.at[i0 + 2]], rows0, sem0)
            pltpu.make_async_copy(y_hbm.at[src_v.at[i0 + 1]], rows1, sem1).wait()
            pltpu.sync_copy(rows1, acc_sh.at[dst_v.at[i0 + 1]], add=True)
            return carry2

        lax.fori_loop(0, IDXSEG // 2 - 1, _pair, 0)
        last = IDXSEG - 2
        pltpu.async_copy(y_hbm.at[src_v.at[last + 1]], rows1, sem1)
        pltpu.make_async_copy(y_hbm.at[src_v.at[last]], rows0, sem0).wait()
        pltpu.sync_copy(rows0, acc_sh.at[dst_v.at[last]], add=True)
        pltpu.make_async_copy(y_hbm.at[src_v.at[last + 1]], rows1, sem1).wait()
        pltpu.sync_copy(rows1, acc_sh.at[dst_v.at[last + 1]], add=True)
        return carry

    lax.fori_loop(0, NSEG, _seg, 0)
    plsc.subcore_barrier()

    pltpu.sync_copy(
        acc_sh.at[pl.ds(s * ROWS_PT, ROWS_PT)],
        out_hbm.at[c, pl.ds(s * ROWS_PT, ROWS_PT)],
    )


@functools.lru_cache(maxsize=1)
def _sc_kernels():
    mesh = plsc.VectorSubcoreMesh(core_axis_name="c", subcore_axis_name="s")
    deg_kernel = pl.kernel(
        _deg_body,
        mesh=mesh,
        out_type=jax.ShapeDtypeStruct((NC, ACC_ROWS, DEG_W), jnp.float32),
        scratch_types=[
            pltpu.VMEM_SHARED((ACC_ROWS, DEG_W), jnp.float32),
            pltpu.VMEM((NCHUNK, CH), jnp.int32),
            pltpu.VMEM((CH, DEG_W), jnp.float32),
            pltpu.VMEM((CH, DEG_W), jnp.float32),
        ],
    )
    agg_kernel = pl.kernel(
        _agg_body,
        mesh=mesh,
        out_type=jax.ShapeDtypeStruct((NC, ACC_ROWS, EMB), jnp.float32),
        scratch_types=[
            pltpu.VMEM_SHARED((ACC_ROWS, EMB), jnp.float32),
            pltpu.VMEM((IDXSEG, CH), jnp.int32),
            pltpu.VMEM((IDXSEG, CH), jnp.int32),
            pltpu.VMEM((CH, EMB), jnp.float32),
            pltpu.VMEM((CH, EMB), jnp.float32),
            pltpu.SemaphoreType.DMA,
            pltpu.SemaphoreType.DMA,
        ],
    )
    return deg_kernel, agg_kernel


def _tc1_body(degp_ref, emb_ref, w1_ref, y1_ref, dinv_ref):
    dp = degp_ref[...]
    deg = dp[0, :N_NODES, 0:1] + dp[1, :N_NODES, 0:1] + 1.0
    dinv = lax.rsqrt(deg)
    dinv_ref[...] = dinv
    xw = jnp.dot(emb_ref[...], w1_ref[...],
                 preferred_element_type=jnp.float32,
                 precision=lax.Precision.HIGHEST)
    y1_ref[...] = xw * dinv


def _tc2_body(aggp_ref, y1_ref, dinv_ref, b1_ref, w2_ref, y2_ref):
    a = aggp_ref[...]
    agg = a[0, :N_NODES, :] + a[1, :N_NODES, :]
    dinv = dinv_ref[...]
    h = jnp.maximum((agg + y1_ref[...]) * dinv + b1_ref[...], 0.0)
    y2 = jnp.dot(h, w2_ref[...],
                 preferred_element_type=jnp.float32,
                 precision=lax.Precision.HIGHEST)
    y2_ref[...] = y2 * dinv


def _tc3_body(aggp_ref, y2_ref, dinv_ref, b2_ref, out_ref):
    a = aggp_ref[...]
    agg = a[0, :N_NODES, :] + a[1, :N_NODES, :]
    out_ref[...] = jnp.maximum(
        (agg + y2_ref[...]) * dinv_ref[...] + b2_ref[...], 0.0)


_tc1 = pl.pallas_call(
    _tc1_body,
    out_shape=[
        jax.ShapeDtypeStruct((N_NODES, EMB), jnp.float32),
        jax.ShapeDtypeStruct((N_NODES, 1), jnp.float32),
    ],
)

_tc2 = pl.pallas_call(
    _tc2_body,
    out_shape=jax.ShapeDtypeStruct((N_NODES, EMB), jnp.float32),
)

_tc3 = pl.pallas_call(
    _tc3_body,
    out_shape=jax.ShapeDtypeStruct((N_NODES, EMB), jnp.float32),
)


def kernel(edge_index, emb, W1, b1, W2, b2):
    src = edge_index[0].astype(jnp.int32)
    dst = edge_index[1].astype(jnp.int32)
    ppw = EPW - N_EDGES // NW
    pad_src = (ppw * jnp.arange(NW, dtype=jnp.int32)[:, None]
               + jnp.arange(ppw, dtype=jnp.int32)[None, :]) % N_NODES
    src = jnp.concatenate([src.reshape(NW, N_EDGES // NW), pad_src], axis=1)
    dummy_rows = (DUMMY + 7 * jnp.arange(NW, dtype=jnp.int32)[:, None]
                  + jnp.arange(ppw, dtype=jnp.int32)[None, :] % 7)
    dst = jnp.concatenate(
        [dst.reshape(NW, N_EDGES // NW), dummy_rows], axis=1)
    src = src.reshape(NW, NCHUNK, CH)
    dst = dst.reshape(NW, NCHUNK, CH)
    b1r = b1.reshape(1, EMB)
    b2r = b2.reshape(1, EMB)

    _deg_kernel, _agg_kernel = _sc_kernels()
    degp = _deg_kernel(dst)
    y1, dinv = _tc1(degp, emb, W1)
    agg1 = _agg_kernel(y1, src, dst)
    y2 = _tc2(agg1, y1, dinv, b1r, W2)
    agg2 = _agg_kernel(y2, src, dst)
    return _tc3(agg2, y2, dinv, b2r)


# speedup vs baseline: 1.0463x; 1.0463x over previous
import functools

import jax
import jax.numpy as jnp
from jax import lax
from jax.experimental import pallas as pl
from jax.experimental.pallas import tpu as pltpu
from jax.experimental.pallas import tpu_sc as plsc

N_NODES = 10000
EMB = 128
N_EDGES = 320000

NC = 2
NS = 16
NW = NC * NS
CH = 128
NCHUNK = 80
IDXSEG = 40
NSEG = NCHUNK // IDXSEG
EPW = NCHUNK * CH
E_PAD = EPW * NW
ROWS_PT = 640
ACC_ROWS = ROWS_PT * NS
DUMMY = N_NODES
DEG_W = 128

def _deg_body(dst_hbm, out_hbm, acc_sh, dst_v, ones_v, zero_v):
    c = lax.axis_index("c")
    s = lax.axis_index("s")
    wid = c * NS + s

    one16 = jnp.full((16,), 1.0, jnp.float32)
    nil16 = jnp.zeros((16,), jnp.float32)

    def _fill(i, carry):
        for j in range(DEG_W // 16):
            ones_v[i, pl.ds(j * 16, 16)] = one16
            zero_v[i, pl.ds(j * 16, 16)] = nil16
        return carry

    lax.fori_loop(0, CH, _fill, 0)

    for k in range(ROWS_PT // CH):
        pltpu.sync_copy(zero_v, acc_sh.at[pl.ds(s * ROWS_PT + k * CH, CH)])

    pltpu.sync_copy(dst_hbm.at[wid], dst_v)
    plsc.subcore_barrier()

    def _chunk(i, carry):
        pltpu.sync_copy(ones_v, acc_sh.at[dst_v.at[i]], add=True)
        return carry

    lax.fori_loop(0, NCHUNK, _chunk, 0)
    plsc.subcore_barrier()

    pltpu.sync_copy(
        acc_sh.at[pl.ds(s * ROWS_PT, ROWS_PT)],
        out_hbm.at[c, pl.ds(s * ROWS_PT, ROWS_PT)],
    )


def _agg_body(y_hbm, src_hbm, dst_hbm, out_hbm, acc_sh,
              src_v, dst_v, rows0, rows1, sem0, sem1):
    c = lax.axis_index("c")
    s = lax.axis_index("s")
    wid = c * NS + s

    nil16 = jnp.zeros((16,), jnp.float32)

    def _zero(i, carry):
        for j in range(EMB // 16):
            rows0[i, pl.ds(j * 16, 16)] = nil16
        return carry

    lax.fori_loop(0, CH, _zero, 0)

    for k in range(ROWS_PT // CH):
        pltpu.sync_copy(rows0, acc_sh.at[pl.ds(s * ROWS_PT + k * CH, CH)])

    plsc.subcore_barrier()

    def _seg(t, carry):
        pltpu.sync_copy(src_hbm.at[wid, pl.ds(t * IDXSEG, IDXSEG)], src_v)
        pltpu.sync_copy(dst_hbm.at[wid, pl.ds(t * IDXSEG, IDXSEG)], dst_v)
        pltpu.async_copy(y_hbm.at[src_v.at[0]], rows0, sem0)

        def _pair(g, carry2):
            i0 = 2 * g
            pltpu.async_copy(y_hbm.at[src_v.at[i0 + 1]], rows1, sem1)
            pltpu.make_async_copy(y_hbm.at[src_v.at[i0]], rows0, sem0).wait()
            pltpu.sync_copy(rows0, acc_sh.at[dst_v.at[i0]], add=True)
            pltpu.async_copy(y_hbm.at[src_v.at[i0 + 2]], rows0, sem0)
            pltpu.make_async_copy(y_hbm.at[src_v.at[i0 + 1]], rows1, sem1).wait()
            pltpu.sync_copy(rows1, acc_sh.at[dst_v.at[i0 + 1]], add=True)
            return carry2

        lax.fori_loop(0, IDXSEG // 2 - 1, _pair, 0)
        last = IDXSEG - 2
        pltpu.async_copy(y_hbm.at[src_v.at[last + 1]], rows1, sem1)
        pltpu.make_async_copy(y_hbm.at[src_v.at[last]], rows0, sem0).wait()
        pltpu.sync_copy(rows0, acc_sh.at[dst_v.at[last]], add=True)
        pltpu.make_async_copy(y_hbm.at[src_v.at[last + 1]], rows1, sem1).wait()
        pltpu.sync_copy(rows1, acc_sh.at[dst_v.at[last + 1]], add=True)
        return carry

    lax.fori_loop(0, NSEG, _seg, 0)
    plsc.subcore_barrier()

    pltpu.sync_copy(
        acc_sh.at[pl.ds(s * ROWS_PT, ROWS_PT)],
        out_hbm.at[c, pl.ds(s * ROWS_PT, ROWS_PT)],
    )


@functools.lru_cache(maxsize=1)
def _sc_kernels():
    mesh = plsc.VectorSubcoreMesh(core_axis_name="c", subcore_axis_name="s")
    deg_kernel = pl.kernel(
        _deg_body,
        mesh=mesh,
        out_type=jax.ShapeDtypeStruct((NC, ACC_ROWS, DEG_W), jnp.float32),
        scratch_types=[
            pltpu.VMEM_SHARED((ACC_ROWS, DEG_W), jnp.float32),
            pltpu.VMEM((NCHUNK, CH), jnp.int32),
            pltpu.VMEM((CH, DEG_W), jnp.float32),
            pltpu.VMEM((CH, DEG_W), jnp.float32),
        ],
    )
    agg_kernel = pl.kernel(
        _agg_body,
        mesh=mesh,
        out_type=jax.ShapeDtypeStruct((NC, ACC_ROWS, EMB), jnp.float32),
        scratch_types=[
            pltpu.VMEM_SHARED((ACC_ROWS, EMB), jnp.float32),
            pltpu.VMEM((IDXSEG, CH), jnp.int32),
            pltpu.VMEM((IDXSEG, CH), jnp.int32),
            pltpu.VMEM((CH, EMB), jnp.float32),
            pltpu.VMEM((CH, EMB), jnp.float32),
            pltpu.SemaphoreType.DMA,
            pltpu.SemaphoreType.DMA,
        ],
    )
    return deg_kernel, agg_kernel


def _tc1_body(degp_ref, emb_ref, w1_ref, y1_ref, dinv_ref):
    dp = degp_ref[...]
    deg = dp[0, :N_NODES, 0:1] + dp[1, :N_NODES, 0:1] + 1.0
    dinv = lax.rsqrt(deg)
    dinv_ref[...] = dinv
    xw = jnp.dot(emb_ref[...], w1_ref[...],
                 preferred_element_type=jnp.float32,
                 precision=lax.Precision.HIGHEST)
    y1_ref[...] = xw * dinv


def _tc2_body(aggp_ref, y1_ref, dinv_ref, b1_ref, w2_ref, y2_ref):
    a = aggp_ref[...]
    agg = a[0, :N_NODES, :] + a[1, :N_NODES, :]
    dinv = dinv_ref[...]
    h = jnp.maximum((agg + y1_ref[...]) * dinv + b1_ref[...], 0.0)
    y2 = jnp.dot(h, w2_ref[...],
                 preferred_element_type=jnp.float32,
                 precision=lax.Precision.HIGHEST)
    y2_ref[...] = y2 * dinv


def _tc3_body(aggp_ref, y2_ref, dinv_ref, b2_ref, out_ref):
    a = aggp_ref[...]
    agg = a[0, :N_NODES, :] + a[1, :N_NODES, :]
    out_ref[...] = jnp.maximum(
        (agg + y2_ref[...]) * dinv_ref[...] + b2_ref[...], 0.0)


_tc1 = pl.pallas_call(
    _tc1_body,
    out_shape=[
        jax.ShapeDtypeStruct((N_NODES, EMB), jnp.float32),
        jax.ShapeDtypeStruct((N_NODES, 1), jnp.float32),
    ],
)

_tc2 = pl.pallas_call(
    _tc2_body,
    out_shape=jax.ShapeDtypeStruct((N_NODES, EMB), jnp.float32),
)

_tc3 = pl.pallas_call(
    _tc3_body,
    out_shape=jax.ShapeDtypeStruct((N_NODES, EMB), jnp.float32),
)


def kernel(edge_index, emb, W1, b1, W2, b2):
    src = edge_index[0].astype(jnp.int32)
    dst = edge_index[1].astype(jnp.int32)
    ppw = EPW - N_EDGES // NW
    pad_src = (ppw * jnp.arange(NW, dtype=jnp.int32)[:, None]
               + jnp.arange(ppw, dtype=jnp.int32)[None, :]) % N_NODES
    src = jnp.concatenate([src.reshape(NW, N_EDGES // NW), pad_src], axis=1)
    dummy_rows = (DUMMY + 7 * jnp.arange(NW, dtype=jnp.int32)[:, None]
                  + jnp.arange(ppw, dtype=jnp.int32)[None, :] % 7)
    dst = jnp.concatenate(
        [dst.reshape(NW, N_EDGES // NW), dummy_rows], axis=1)
    src = src.reshape(NW, NCHUNK, CH)
    dst = dst.reshape(NW, NCHUNK, CH)
    b1r = b1.reshape(1, EMB)
    b2r = b2.reshape(1, EMB)

    _deg_kernel, _agg_kernel = _sc_kernels()
    degp = _deg_kernel(dst)
    y1, dinv = _tc1(degp, emb, W1)
    agg1 = _agg_kernel(y1, src, dst)
    y2 = _tc2(agg1, y1, dinv, b1r, W2)
    agg2 = _agg_kernel(y2, src, dst)
    return _tc3(agg2, y2, dinv, b2r)
